# baseline (speedup 1.0000x reference)
import functools

import jax
import jax.numpy as jnp
import numpy as np
from jax.experimental import pallas as pl
from jax.experimental.pallas import tpu as pltpu

K = 1024
CD = 10
D = 64
SCALE = 1.0
INV_TEMP = 100.0
ENT_W = 0.1
COMMIT_W = 0.25
GAMMA = 1.0
B, N = 8, 4096
TOKENS = B * N
BPG = 4
NSPLIT = 2
CHN = N // NSPLIT

_s = 2.0 * INV_TEMP * SCALE
_CTAT = np.zeros((136, 2 * CD), dtype=np.float32)
for _d in range(7):
    _J = np.arange(128)
    _CTAT[:128, _d] = _s * (2.0 * ((_J >> (6 - _d)) & 1) - 1.0)
    _CTAT[:128, CD + _d] = -_s
for _d in range(7, CD):
    _L = np.arange(8)
    _CTAT[128:136, _d] = _s * (2.0 * ((_L >> (9 - _d)) & 1) - 1.0)
    _CTAT[128:136, CD + _d] = -_s

_IMASK = (2 ** np.arange(CD - 1, -1, -1)).astype(np.float32)


def _lfq_body(z_ref, wi_ref, bi_ref, wo_ref, bo_ref, cta_ref,
              out_ref, idx_ref, aux_ref,
              avg_acc, sums_acc):
    b = pl.program_id(0)

    @pl.when(b == 0)
    def _init():
        avg_acc[...] = jnp.zeros_like(avg_acc)
        sums_acc[0] = 0.0
        sums_acc[1] = 0.0

    ent_tile = 0.0
    commit_tile = 0.0
    avg_upd = jnp.zeros((128, 8), dtype=jnp.float32)
    for bb in range(BPG):
      for h in range(NSPLIT):
        sl = slice(h * CHN, (h + 1) * CHN)
        zt = z_ref[bb, :, sl]
        xt = jax.lax.dot_general(wi_ref[...], zt, (((1,), (0,)), ((), ())),
                                 preferred_element_type=jnp.float32) + bi_ref[...]
        qt = jnp.where(xt > 0, SCALE, -SCALE).astype(jnp.float32)
        axt = jnp.abs(xt)

        y2 = jax.lax.dot_general(wo_ref[...], qt, (((1,), (0,)), ((), ())),
                                 preferred_element_type=jnp.float32)
        out_ref[bb, :, sl] = y2[:D, :] + bo_ref[...]
        idxf = y2[D:D + 1, :] + (float(K) - 1.0) * 0.5
        idx_ref[pl.ds(b * BPG + bb, 1), sl] = idxf.astype(jnp.int32)

        commit_tile += jnp.sum((xt - qt) ** 2)

        xa = jnp.concatenate([xt, axt], axis=0)
        lp = jax.lax.dot_general(cta_ref[...], xa, (((1,), (0,)), ((), ())),
                                 preferred_element_type=jnp.float32)
        e7 = jnp.exp(lp[:128, :])
        e3 = jnp.exp(lp[128:136, :])
        s7 = jnp.sum(e7, axis=0, keepdims=True)
        s3 = jnp.sum(e3, axis=0, keepdims=True)
        s73 = s7 * s3
        p3s = e3 * (1.0 / s73)
        p3t = jnp.swapaxes(p3s, 0, 1)
        avg_upd += jax.lax.dot_general(e7, p3t, (((1,), (0,)), ((), ())),
                                       preferred_element_type=jnp.float32)

        u = (2.0 * _s) * axt
        a = jnp.exp(-u)
        g = u * a / (1.0 + a)
        ent_tile += jnp.sum(g) + jnp.sum(jnp.log(s73))

    avg_acc[...] += avg_upd
    sums_acc[0] += ent_tile
    sums_acc[1] += commit_tile

    @pl.when(b == B // BPG - 1)
    def _fin():
        nt = float(TOKENS)
        pse = sums_acc[0] / nt
        ap = avg_acc[...] / nt
        ce = jnp.sum(-ap * jnp.log(jnp.clip(ap, 1e-20, None)))
        commit = sums_acc[1] / (nt * CD)
        aux = (pse - GAMMA * ce) * ENT_W + COMMIT_W * commit
        aux_ref[...] = jnp.reshape(aux, (1, 1))


@functools.partial(jax.jit, static_argnames=())
def kernel(z_e_x, W_in, b_in, W_out, b_out):
    zt = jnp.swapaxes(z_e_x, 1, 2)
    bi = b_in.reshape(CD, 1)
    bo = b_out.reshape(D, 1)
    cta = jnp.asarray(_CTAT)
    wo_aug = jnp.concatenate([W_out, jnp.asarray(_IMASK)[None, :] * 0.5], axis=0)

    out_t, idx, aux = pl.pallas_call(
        _lfq_body,
        grid=(B // BPG,),
        in_specs=[
            pl.BlockSpec((BPG, D, N), lambda b: (b, 0, 0)),
            pl.BlockSpec((CD, D), lambda b: (0, 0)),
            pl.BlockSpec((CD, 1), lambda b: (0, 0)),
            pl.BlockSpec((D + 1, CD), lambda b: (0, 0)),
            pl.BlockSpec((D, 1), lambda b: (0, 0)),
            pl.BlockSpec((136, 2 * CD), lambda b: (0, 0)),
        ],
        out_specs=[
            pl.BlockSpec((BPG, D, N), lambda b: (b, 0, 0)),
            pl.BlockSpec((B, N), lambda b: (0, 0)),
            pl.BlockSpec((1, 1), lambda b: (0, 0)),
        ],
        out_shape=[
            jax.ShapeDtypeStruct((B, D, N), jnp.float32),
            jax.ShapeDtypeStruct((B, N), jnp.int32),
            jax.ShapeDtypeStruct((1, 1), jnp.float32),
        ],
        scratch_shapes=[
            pltpu.VMEM((128, 8), jnp.float32),
            pltpu.SMEM((2,), jnp.float32),
        ],
    )(zt, W_in, bi, wo_aug, bo, cta)

    out = jnp.swapaxes(out_t, 1, 2)
    return (out, idx, aux.reshape(()))


# speedup vs baseline: 1.1230x; 1.1230x over previous
import functools

import jax
import jax.numpy as jnp
import numpy as np
from jax.experimental import pallas as pl
from jax.experimental.pallas import tpu as pltpu

K = 1024
CD = 10
D = 64
SCALE = 1.0
INV_TEMP = 100.0
ENT_W = 0.1
COMMIT_W = 0.25
GAMMA = 1.0
B, N = 8, 4096
TOKENS = B * N
BPG = 2
NSPLIT = 1
CHN = N // NSPLIT

_s = 2.0 * INV_TEMP * SCALE
_CTAT = np.zeros((136, 2 * CD), dtype=np.float32)
for _d in range(7):
    _J = np.arange(128)
    _CTAT[:128, _d] = _s * (2.0 * ((_J >> (6 - _d)) & 1) - 1.0)
    _CTAT[:128, CD + _d] = -_s
for _d in range(7, CD):
    _L = np.arange(8)
    _CTAT[128:136, _d] = _s * (2.0 * ((_L >> (9 - _d)) & 1) - 1.0)
    _CTAT[128:136, CD + _d] = -_s

_IMASK = (2 ** np.arange(CD - 1, -1, -1)).astype(np.float32)


def _lfq_body(z_ref, wi_ref, bi_ref, wo_ref, bo_ref, cta_ref,
              out_ref, idx_ref, aux_ref,
              avg_acc, sums_acc):
    b = pl.program_id(0)

    @pl.when(b == 0)
    def _init():
        avg_acc[...] = jnp.zeros_like(avg_acc)
        sums_acc[0] = 0.0
        sums_acc[1] = 0.0

    ent_tile = 0.0
    commit_tile = 0.0
    avg_upd = jnp.zeros((128, 8), dtype=jnp.float32)
    for bb in range(BPG):
      for h in range(NSPLIT):
        sl = slice(h * CHN, (h + 1) * CHN)
        zt = z_ref[bb, :, sl]
        xt = jax.lax.dot_general(wi_ref[...], zt, (((1,), (0,)), ((), ())),
                                 preferred_element_type=jnp.float32) + bi_ref[...]
        qt = jnp.where(xt > 0, SCALE, -SCALE).astype(jnp.float32)
        axt = jnp.abs(xt)

        y2 = jax.lax.dot_general(wo_ref[...], qt, (((1,), (0,)), ((), ())),
                                 preferred_element_type=jnp.float32)
        out_ref[bb, :, sl] = y2[:D, :] + bo_ref[...]
        idxf = y2[D:D + 1, :] + (float(K) - 1.0) * 0.5
        idx_ref[pl.ds(b * BPG + bb, 1), sl] = idxf.astype(jnp.int32)

        commit_tile += jnp.sum((xt - qt) ** 2)

        xa = jnp.concatenate([xt, axt], axis=0)
        lp = jax.lax.dot_general(cta_ref[...], xa, (((1,), (0,)), ((), ())),
                                 preferred_element_type=jnp.float32)
        e7 = jnp.exp(lp[:128, :])
        e3 = jnp.exp(lp[128:136, :])
        s7 = jnp.sum(e7, axis=0, keepdims=True)
        s3 = jnp.sum(e3, axis=0, keepdims=True)
        s73 = s7 * s3
        p3s = e3 * (1.0 / s73)
        p3t = jnp.swapaxes(p3s, 0, 1)
        avg_upd += jax.lax.dot_general(e7, p3t, (((1,), (0,)), ((), ())),
                                       preferred_element_type=jnp.float32)

        u = (2.0 * _s) * axt
        a = jnp.exp(-u)
        g = u * a / (1.0 + a)
        ent_tile += jnp.sum(g) + jnp.sum(jnp.log(s73))

    avg_acc[...] += avg_upd
    sums_acc[0] += ent_tile
    sums_acc[1] += commit_tile

    @pl.when(b == B // BPG - 1)
    def _fin():
        nt = float(TOKENS)
        pse = sums_acc[0] / nt
        ap = avg_acc[...] / nt
        ce = jnp.sum(-ap * jnp.log(jnp.clip(ap, 1e-20, None)))
        commit = sums_acc[1] / (nt * CD)
        aux = (pse - GAMMA * ce) * ENT_W + COMMIT_W * commit
        aux_ref[...] = jnp.reshape(aux, (1, 1))


@functools.partial(jax.jit, static_argnames=())
def kernel(z_e_x, W_in, b_in, W_out, b_out):
    zt = jnp.swapaxes(z_e_x, 1, 2)
    bi = b_in.reshape(CD, 1)
    bo = b_out.reshape(D, 1)
    cta = jnp.asarray(_CTAT)
    wo_aug = jnp.concatenate([W_out, jnp.asarray(_IMASK)[None, :] * 0.5], axis=0)

    out_t, idx, aux = pl.pallas_call(
        _lfq_body,
        grid=(B // BPG,),
        in_specs=[
            pl.BlockSpec((BPG, D, N), lambda b: (b, 0, 0)),
            pl.BlockSpec((CD, D), lambda b: (0, 0)),
            pl.BlockSpec((CD, 1), lambda b: (0, 0)),
            pl.BlockSpec((D + 1, CD), lambda b: (0, 0)),
            pl.BlockSpec((D, 1), lambda b: (0, 0)),
            pl.BlockSpec((136, 2 * CD), lambda b: (0, 0)),
        ],
        out_specs=[
            pl.BlockSpec((BPG, D, N), lambda b: (b, 0, 0)),
            pl.BlockSpec((B, N), lambda b: (0, 0)),
            pl.BlockSpec((1, 1), lambda b: (0, 0)),
        ],
        out_shape=[
            jax.ShapeDtypeStruct((B, D, N), jnp.float32),
            jax.ShapeDtypeStruct((B, N), jnp.int32),
            jax.ShapeDtypeStruct((1, 1), jnp.float32),
        ],
        scratch_shapes=[
            pltpu.VMEM((128, 8), jnp.float32),
            pltpu.SMEM((2,), jnp.float32),
        ],
    )(zt, W_in, bi, wo_aug, bo, cta)

    out = jnp.swapaxes(out_t, 1, 2)
    return (out, idx, aux.reshape(()))
